# Initial kernel scaffold; baseline (speedup 1.0000x reference)
import jax
import jax.numpy as jnp
from jax.experimental import pallas as pl


def kernel(im, coord, noise, W_sparse, b_sparse, init_sky, W_out, b_out):
    raise NotImplementedError("write your pallas kernel here")


# speedup vs baseline: 1.5622x; 1.5622x over previous
import functools

import jax
import jax.numpy as jnp
from jax import lax
from jax.experimental import pallas as pl
from jax.experimental.pallas import tpu as pltpu
from jax.experimental.pallas import tpu_sc as plsc

_F32 = jnp.float32
_I32 = jnp.int32

_N = 4 * 256 * 256
_M = _N
_P = 2048
_NPAD = _N + _P

_NSUB = 16
_NCORE = 2
_SLICE = _N // _NSUB
_RBITS = 13
_R = 1 << _RBITS
_PASSES = _M // (_R * _NCORE)
_NBINS = _PASSES * _NCORE
_G = 128
_TRASH = _R
_SHARE = 520
_SROWS = _NSUB * _SHARE
_ZB = 20
_STCAP = _SLICE + (_NBINS + 1) * _G
_CH = 2048
_PKPAD = (_TRASH << 18) | 0


def _min_kernel(cx, cy, cz, o_ref):
    step = pl.program_id(0)

    @pl.when(step == 0)
    def _():
        o_ref[...] = jnp.full_like(o_ref[...], jnp.inf)

    x = cx[...]
    y = cy[...]
    z = cz[...]
    r = jnp.sqrt(x * x + y * y + z * z)
    city = r < 600.0
    big = jnp.float32(jnp.inf)
    mx = jnp.min(jnp.where(city, x * 10.0, big), axis=0)
    my = jnp.min(jnp.where(city, y * 10.0, big), axis=0)
    mz = jnp.min(jnp.where(city, z * 10.0, big), axis=0)
    o_ref[0, :] = jnp.minimum(o_ref[0, :], mx)
    o_ref[1, :] = jnp.minimum(o_ref[1, :], my)
    o_ref[2, :] = jnp.minimum(o_ref[2, :], mz)


def _seg_kernel(minp, cx, cy, cz, seg_ref):
    bx = jnp.floor(jnp.min(minp[0, :]))
    by = jnp.floor(jnp.min(minp[1, :]))
    bz = jnp.floor(jnp.min(minp[2, :]))
    x = cx[...]
    y = cy[...]
    z = cz[...]
    r = jnp.sqrt(x * x + y * y + z * z)
    city = r < 600.0
    lx = jnp.floor(x * 10.0 - bx).astype(_I32)
    ly = jnp.floor(y * 10.0 - by).astype(_I32)
    lz = jnp.floor(z * 10.0 - bz).astype(_I32)
    h = lx * 73856093 + ly * 19349663 + lz * 83492791
    vid = jnp.bitwise_and(h, _M - 1)
    seg_ref[...] = jnp.where(city, vid, _M)


def _pre_kernel(imf, c0b, noise, Ws, bs, Wo, bo,
                feat_ref, ostat_ref):
    cvec = jnp.dot(noise[...], Ws[3:, :],
                   preferred_element_type=_F32) + bs[...][None, :]
    feat = jnp.dot(imf[...], Ws[:3, :], preferred_element_type=_F32) + cvec
    feat = jnp.maximum(feat, 0.0)
    p = feat.shape[0]
    tail = jnp.concatenate(
        [jnp.ones((p, 1), _F32), jnp.zeros((p, 63), _F32)], axis=1)
    feat_ref[...] = jnp.concatenate([feat, tail], axis=1)

    stat = (jnp.dot(imf[...], Wo[0:3, :], preferred_element_type=_F32)
            + jnp.dot(c0b[...], Wo[67:70, :], preferred_element_type=_F32)
            + jnp.dot(noise[...], Wo[70:86, :], preferred_element_type=_F32)
            + bo[...][None, :])
    ostat_ref[...] = stat


def _post_kernel(c0b, feat, psum, Wo, sky, ostat, o_ref):
    ps = psum[...]
    cnt2 = jnp.maximum(ps[:, 64:65], 1.0)
    pooled = ps[:, 0:64] / cnt2
    sp = jnp.maximum(feat[:, 0:64] + pooled, 0.0)
    c_city = jnp.dot(sp, Wo[3:67, :], preferred_element_type=_F32)
    sky3 = jnp.dot(sky[...][None, :], Wo[3:67, :],
                   preferred_element_type=_F32)
    cb = c0b[...]
    r2 = jnp.sum(cb * cb, axis=1, keepdims=True)
    city2 = jnp.sqrt(r2) < 600.0
    o_ref[...] = ostat[...] + jnp.where(city2, c_city, sky3)


def _read48(ref, idx):
    tot = jnp.zeros((), _I32)
    for k in range(3):
        v = ref[pl.ds(k * 16, 16)]
        lane = lax.iota(_I32, 16) + k * 16
        tot = tot + jnp.sum(jnp.where(lane == idx, v, 0))
    return tot


def _sc_body(feat_hbm, seg_hbm, psum_hbm,
             segc_v, st_v, pt_v, bk_v, row_v, zb_v,
             hist_v, start_v, run_v, sums_sh, sem):
    c = lax.axis_index("c")
    s = lax.axis_index("s")
    base = s * _SLICE
    lanes = lax.iota(_I32, 16)

    for rr in range(_ZB):
        for k in range(8):
            zb_v[rr, pl.ds(k * 16, 16)] = jnp.zeros((16,), _F32)
    for k in range(3):
        hist_v[pl.ds(k * 16, 16)] = jnp.zeros((16,), _I32)

    def prefill(g, t):
        flat = g * 16 + lanes
        plsc.store_scatter(st_v, [flat], jnp.full((16,), _PKPAD, _I32))
        return t

    lax.fori_loop(0, _STCAP // 16, prefill, 0)

    def hist_chunk(ch, t):
        pltpu.sync_copy(seg_hbm.at[pl.ds(base + ch * _CH, _CH)], segc_v)

        def hist_body(g, u):
            sv = segc_v[pl.ds(g * 16, 16)]
            b = lax.shift_right_logical(sv, _RBITS)
            offv = plsc.load_gather(hist_v, [b])
            cnt, last = plsc.scan_count(b)
            plsc.store_scatter(hist_v, [b], offv + cnt, mask=last)
            return u

        return lax.fori_loop(0, _CH // 16, hist_body, t)

    lax.fori_loop(0, _SLICE // _CH, hist_chunk, 0)

    carry = jnp.zeros((), _I32)
    for k in range(3):
        hv = hist_v[pl.ds(k * 16, 16)]
        av = jnp.bitwise_and(hv + (_G - 1), ~(_G - 1))
        cs = jnp.cumsum(av)
        start_v[pl.ds(k * 16, 16)] = cs - av + carry
        run_v[pl.ds(k * 16, 16)] = cs - av + carry
        carry = carry + jnp.sum(av)

    def perm_chunk(ch, t):
        pltpu.sync_copy(seg_hbm.at[pl.ds(base + ch * _CH, _CH)], segc_v)

        def perm_body(g, u):
            sv = segc_v[pl.ds(g * 16, 16)]
            b = lax.shift_right_logical(sv, _RBITS)
            offv = plsc.load_gather(run_v, [b])
            cnt, last = plsc.scan_count(b)
            pos = offv + cnt - 1
            plsc.store_scatter(run_v, [b], offv + cnt, mask=last)
            pk = (lax.shift_left(jnp.bitwise_and(sv, _R - 1), 18)
                  | (base + ch * _CH + g * 16 + lanes))
            plsc.store_scatter(st_v, [pos], pk)
            return u

        return lax.fori_loop(0, _CH // 16, perm_body, t)

    lax.fori_loop(0, _SLICE // _CH, perm_chunk, 0)

    def pass_body(p, t):
        row0 = s * _SHARE

        def zloop(j, u):
            pltpu.sync_copy(zb_v, sums_sh.at[pl.ds(row0 + j * _ZB, _ZB)])
            return u

        lax.fori_loop(0, _SHARE // _ZB, zloop, 0)
        plsc.subcore_barrier()

        bin_id = 2 * p + c
        s_b = _read48(start_v, bin_id)
        n_b = _read48(hist_v, bin_id)
        ng = (n_b + _G - 1) // _G

        def unpack(g, slot):
            for k in range(8):
                pk = st_v[pl.ds(s_b + g * _G + k * 16, 16)]
                pt_v[slot, pl.ds(k * 16, 16)] = (
                    jnp.bitwise_and(pk, (1 << 18) - 1))
                bk_v[slot, pl.ds(k * 16, 16)] = (
                    lax.shift_right_logical(pk, 18))

        def stage_a(g, u):
            unpack(g, 0)
            pltpu.async_copy(feat_hbm.at[pt_v.at[0]], row_v, sem).wait()
            pltpu.sync_copy(row_v, sums_sh.at[bk_v.at[0]], add=True)
            return u

        lax.fori_loop(0, ng, stage_a, 0)
        plsc.subcore_barrier()

        def stage_b(g, u):
            unpack(g, 1)
            for k in range(8):
                pt = pt_v[1, pl.ds(k * 16, 16)]
                bk = bk_v[1, pl.ds(k * 16, 16)]
                pt_v[1, pl.ds(k * 16, 16)] = (
                    jnp.where(bk == _TRASH, _N, pt))
            pltpu.async_copy(sums_sh.at[bk_v.at[1]], row_v, sem).wait()
            pltpu.sync_copy(row_v, psum_hbm.at[pt_v.at[1]])
            return u

        lax.fori_loop(0, ng, stage_b, 0)
        plsc.subcore_barrier()
        return t

    lax.fori_loop(0, _PASSES, pass_body, 0)


def _segment_pool(featm, seg):
    mesh = plsc.VectorSubcoreMesh(core_axis_name="c", subcore_axis_name="s")
    fn = functools.partial(
        pl.kernel,
        mesh=mesh,
        compiler_params=pltpu.CompilerParams(needs_layout_passes=False),
        out_type=[
            jax.ShapeDtypeStruct((_NPAD, 128), _F32),
        ],
        scratch_types=[
            pltpu.VMEM((_CH,), _I32),
            pltpu.VMEM((_STCAP,), _I32),
            pltpu.VMEM((2, _G), _I32),
            pltpu.VMEM((2, _G), _I32),
            pltpu.VMEM((_G, 128), _F32),
            pltpu.VMEM((_ZB, 128), _F32),
            pltpu.VMEM((48,), _I32),
            pltpu.VMEM((48,), _I32),
            pltpu.VMEM((48,), _I32),
            pltpu.VMEM_SHARED((_SROWS, 128), _F32),
            pltpu.SemaphoreType.DMA,
        ],
    )(_sc_body)
    return fn(featm, seg)


def kernel(im, coord, noise, W_sparse, b_sparse, init_sky, W_out, b_out):
    B, F, C, H, W = im.shape
    n = F * H * W
    assert n == _N

    c0 = coord[0]
    im_ft = jnp.transpose(im[:, :, :3], (0, 1, 3, 4, 2)).reshape(n, 3)

    cx = c0[:, 0].reshape(n // 128, 128)
    cy = c0[:, 1].reshape(n // 128, 128)
    cz = c0[:, 2].reshape(n // 128, 128)
    blk = 256
    minp = pl.pallas_call(
        _min_kernel,
        grid=(n // 128 // blk,),
        in_specs=[pl.BlockSpec((blk, 128), lambda i: (i, 0))] * 3,
        out_specs=pl.BlockSpec((8, 128), lambda i: (0, 0)),
        out_shape=jax.ShapeDtypeStruct((8, 128), _F32),
    )(cx, cy, cz)

    seg2d = pl.pallas_call(
        _seg_kernel,
        grid=(n // 128 // blk,),
        in_specs=[
            pl.BlockSpec((8, 128), lambda i: (0, 0)),
            pl.BlockSpec((blk, 128), lambda i: (i, 0)),
            pl.BlockSpec((blk, 128), lambda i: (i, 0)),
            pl.BlockSpec((blk, 128), lambda i: (i, 0)),
        ],
        out_specs=pl.BlockSpec((blk, 128), lambda i: (i, 0)),
        out_shape=jax.ShapeDtypeStruct((n // 128, 128), _I32),
    )(minp, cx, cy, cz)
    seg = seg2d.reshape(n)

    imf_p = jnp.concatenate([im_ft, jnp.zeros((_P, 3), _F32)], axis=0)
    c0_p = jnp.concatenate([c0, jnp.zeros((_P, 3), _F32)], axis=0)
    nblk = _NPAD // _P
    featm, ostat = pl.pallas_call(
        _pre_kernel,
        grid=(nblk,),
        in_specs=[
            pl.BlockSpec((_P, 3), lambda i: (i, 0)),
            pl.BlockSpec((_P, 3), lambda i: (i, 0)),
            pl.BlockSpec((1, 16), lambda i: (0, 0)),
            pl.BlockSpec((19, 64), lambda i: (0, 0)),
            pl.BlockSpec((64,), lambda i: (0,)),
            pl.BlockSpec((86, 3), lambda i: (0, 0)),
            pl.BlockSpec((3,), lambda i: (0,)),
        ],
        out_specs=[
            pl.BlockSpec((_P, 128), lambda i: (i, 0)),
            pl.BlockSpec((_P, 3), lambda i: (i, 0)),
        ],
        out_shape=[
            jax.ShapeDtypeStruct((_NPAD, 128), _F32),
            jax.ShapeDtypeStruct((_NPAD, 3), _F32),
        ],
    )(imf_p, c0_p, noise, W_sparse, b_sparse, W_out, b_out)

    (psum,) = _segment_pool(featm, seg)

    out = pl.pallas_call(
        _post_kernel,
        grid=(n // _P,),
        in_specs=[
            pl.BlockSpec((_P, 3), lambda i: (i, 0)),
            pl.BlockSpec((_P, 128), lambda i: (i, 0)),
            pl.BlockSpec((_P, 128), lambda i: (i, 0)),
            pl.BlockSpec((86, 3), lambda i: (0, 0)),
            pl.BlockSpec((64,), lambda i: (0,)),
            pl.BlockSpec((_P, 3), lambda i: (i, 0)),
        ],
        out_specs=pl.BlockSpec((_P, 3), lambda i: (i, 0)),
        out_shape=jax.ShapeDtypeStruct((n, 3), _F32),
    )(c0_p, featm, psum, W_out, init_sky, ostat)
    return out


# speedup vs baseline: 1.6787x; 1.0746x over previous
import functools

import jax
import jax.numpy as jnp
from jax import lax
from jax.experimental import pallas as pl
from jax.experimental.pallas import tpu as pltpu
from jax.experimental.pallas import tpu_sc as plsc

_F32 = jnp.float32
_I32 = jnp.int32

_N = 4 * 256 * 256
_M = _N
_P = 2048
_NPAD = _N + _P

_NSUB = 16
_NCORE = 2
_SLICE = _N // _NSUB
_RBITS = 13
_R = 1 << _RBITS
_PASSES = _M // (_R * _NCORE)
_NBINS = _PASSES * _NCORE
_G = 128
_TRASH = _R
_SHARE = 520
_SROWS = _NSUB * _SHARE
_ZB = 20
_STCAP = _SLICE + (_NBINS + 1) * _G
_CH = 2048
_PKPAD = (_TRASH << 18) | 0


def _min_kernel(cx, cy, cz, o_ref):
    step = pl.program_id(0)

    @pl.when(step == 0)
    def _():
        o_ref[...] = jnp.full_like(o_ref[...], jnp.inf)

    x = cx[...]
    y = cy[...]
    z = cz[...]
    r = jnp.sqrt(x * x + y * y + z * z)
    city = r < 600.0
    big = jnp.float32(jnp.inf)
    mx = jnp.min(jnp.where(city, x * 10.0, big), axis=0)
    my = jnp.min(jnp.where(city, y * 10.0, big), axis=0)
    mz = jnp.min(jnp.where(city, z * 10.0, big), axis=0)
    o_ref[0, :] = jnp.minimum(o_ref[0, :], mx)
    o_ref[1, :] = jnp.minimum(o_ref[1, :], my)
    o_ref[2, :] = jnp.minimum(o_ref[2, :], mz)


def _seg_kernel(minp, cx, cy, cz, seg_ref):
    bx = jnp.floor(jnp.min(minp[0, :]))
    by = jnp.floor(jnp.min(minp[1, :]))
    bz = jnp.floor(jnp.min(minp[2, :]))
    x = cx[...]
    y = cy[...]
    z = cz[...]
    r = jnp.sqrt(x * x + y * y + z * z)
    city = r < 600.0
    lx = jnp.floor(x * 10.0 - bx).astype(_I32)
    ly = jnp.floor(y * 10.0 - by).astype(_I32)
    lz = jnp.floor(z * 10.0 - bz).astype(_I32)
    h = lx * 73856093 + ly * 19349663 + lz * 83492791
    vid = jnp.bitwise_and(h, _M - 1)
    seg_ref[...] = jnp.where(city, vid, _M)


def _feat_mlp(imf, noise, Ws, bs):
    cvec = jnp.dot(noise, Ws[3:, :],
                   preferred_element_type=_F32) + bs[None, :]
    feat = jnp.dot(imf, Ws[:3, :], preferred_element_type=_F32) + cvec
    return jnp.maximum(feat, 0.0)


def _pre_kernel(imf, noise, Ws, bs, feat_ref):
    feat = _feat_mlp(imf[...], noise[...], Ws[...], bs[...])
    p = feat.shape[0]
    tail = jnp.concatenate(
        [jnp.ones((p, 1), _F32), jnp.zeros((p, 63), _F32)], axis=1)
    feat_ref[...] = jnp.concatenate([feat, tail], axis=1)


def _post_kernel(c0b, imf, noise, Ws, bs, psum, Wo, bo, sky, o_ref):
    feat = _feat_mlp(imf[...], noise[...], Ws[...], bs[...])
    ps = psum[...]
    cnt2 = jnp.maximum(ps[:, 64:65], 1.0)
    pooled = ps[:, 0:64] / cnt2
    sp = jnp.maximum(feat + pooled, 0.0)
    c_city = jnp.dot(sp, Wo[3:67, :], preferred_element_type=_F32)
    sky3 = jnp.dot(sky[...][None, :], Wo[3:67, :],
                   preferred_element_type=_F32)
    cb = c0b[...]
    stat = (jnp.dot(imf[...], Wo[0:3, :], preferred_element_type=_F32)
            + jnp.dot(cb, Wo[67:70, :], preferred_element_type=_F32)
            + jnp.dot(noise[...], Wo[70:86, :], preferred_element_type=_F32)
            + bo[...][None, :])
    r2 = jnp.sum(cb * cb, axis=1, keepdims=True)
    city2 = jnp.sqrt(r2) < 600.0
    o_ref[...] = stat + jnp.where(city2, c_city, sky3)


def _read48(ref, idx):
    tot = jnp.zeros((), _I32)
    for k in range(3):
        v = ref[pl.ds(k * 16, 16)]
        lane = lax.iota(_I32, 16) + k * 16
        tot = tot + jnp.sum(jnp.where(lane == idx, v, 0))
    return tot


def _sc_body(feat_hbm, seg_hbm, psum_hbm,
             segc_v, st_v, pt_v, bk_v, row_v, row2_v, zb_v,
             hist_v, start_v, run_v, sums_sh, sem, sem2, sem3):
    c = lax.axis_index("c")
    s = lax.axis_index("s")
    base = s * _SLICE
    lanes = lax.iota(_I32, 16)

    for rr in range(_ZB):
        for k in range(8):
            zb_v[rr, pl.ds(k * 16, 16)] = jnp.zeros((16,), _F32)
    for k in range(3):
        hist_v[pl.ds(k * 16, 16)] = jnp.zeros((16,), _I32)

    def prefill(g, t):
        flat = g * 16 + lanes
        plsc.store_scatter(st_v, [flat], jnp.full((16,), _PKPAD, _I32))
        return t

    lax.fori_loop(0, _STCAP // 16, prefill, 0)

    def hist_chunk(ch, t):
        pltpu.sync_copy(seg_hbm.at[pl.ds(base + ch * _CH, _CH)], segc_v)

        def hist_body(g, u):
            sv = segc_v[pl.ds(g * 16, 16)]
            b = lax.shift_right_logical(sv, _RBITS)
            offv = plsc.load_gather(hist_v, [b])
            cnt, last = plsc.scan_count(b)
            plsc.store_scatter(hist_v, [b], offv + cnt, mask=last)
            return u

        return lax.fori_loop(0, _CH // 16, hist_body, t)

    lax.fori_loop(0, _SLICE // _CH, hist_chunk, 0)

    carry = jnp.zeros((), _I32)
    for k in range(3):
        hv = hist_v[pl.ds(k * 16, 16)]
        av = jnp.bitwise_and(hv + (_G - 1), ~(_G - 1))
        cs = jnp.cumsum(av)
        start_v[pl.ds(k * 16, 16)] = cs - av + carry
        run_v[pl.ds(k * 16, 16)] = cs - av + carry
        carry = carry + jnp.sum(av)

    def perm_chunk(ch, t):
        pltpu.sync_copy(seg_hbm.at[pl.ds(base + ch * _CH, _CH)], segc_v)

        def perm_body(g, u):
            sv = segc_v[pl.ds(g * 16, 16)]
            b = lax.shift_right_logical(sv, _RBITS)
            offv = plsc.load_gather(run_v, [b])
            cnt, last = plsc.scan_count(b)
            pos = offv + cnt - 1
            plsc.store_scatter(run_v, [b], offv + cnt, mask=last)
            pk = (lax.shift_left(jnp.bitwise_and(sv, _R - 1), 18)
                  | (base + ch * _CH + g * 16 + lanes))
            plsc.store_scatter(st_v, [pos], pk)
            return u

        return lax.fori_loop(0, _CH // 16, perm_body, t)

    lax.fori_loop(0, _SLICE // _CH, perm_chunk, 0)

    def pass_body(p, t):
        row0 = s * _SHARE

        def zfire(j, u):
            pltpu.async_copy(zb_v, sums_sh.at[pl.ds(row0 + j * _ZB, _ZB)],
                             sem3)
            return u

        lax.fori_loop(0, _SHARE // _ZB, zfire, 0)

        def zdrain(j, u):
            pltpu.make_async_copy(
                zb_v, sums_sh.at[pl.ds(row0 + j * _ZB, _ZB)], sem3).wait()
            return u

        lax.fori_loop(0, _SHARE // _ZB, zdrain, 0)
        plsc.subcore_barrier()

        bin_id = 2 * p + c
        s_b = _read48(start_v, bin_id)
        n_b = _read48(hist_v, bin_id)
        ng = (n_b + _G - 1) // _G
        nh = ng // 2

        def unpack(g, slot, remap=False):
            for k in range(8):
                pk = st_v[pl.ds(s_b + g * _G + k * 16, 16)]
                pt = jnp.bitwise_and(pk, (1 << 18) - 1)
                bk = lax.shift_right_logical(pk, 18)
                if remap:
                    pt = jnp.where(bk == _TRASH, _N, pt)
                pt_v[slot, pl.ds(k * 16, 16)] = pt
                bk_v[slot, pl.ds(k * 16, 16)] = bk

        def pair_a(h, u):
            g0 = 2 * h
            unpack(g0, 0)
            cp0 = pltpu.async_copy(feat_hbm.at[pt_v.at[0]], row_v, sem)
            unpack(g0 + 1, 1)
            cp1 = pltpu.async_copy(feat_hbm.at[pt_v.at[1]], row2_v, sem2)
            cp0.wait()
            pltpu.sync_copy(row_v, sums_sh.at[bk_v.at[0]], add=True)
            cp1.wait()
            pltpu.sync_copy(row2_v, sums_sh.at[bk_v.at[1]], add=True)
            return u

        lax.fori_loop(0, nh, pair_a, 0)

        @pl.when(ng > 2 * nh)
        def _():
            unpack(2 * nh, 0)
            pltpu.async_copy(feat_hbm.at[pt_v.at[0]], row_v, sem).wait()
            pltpu.sync_copy(row_v, sums_sh.at[bk_v.at[0]], add=True)

        plsc.subcore_barrier()

        def pair_b(h, u):
            g0 = 2 * h
            unpack(g0, 0, remap=True)
            cp0 = pltpu.async_copy(sums_sh.at[bk_v.at[0]], row_v, sem)
            unpack(g0 + 1, 1, remap=True)
            cp1 = pltpu.async_copy(sums_sh.at[bk_v.at[1]], row2_v, sem2)
            cp0.wait()
            w0 = pltpu.async_copy(row_v, psum_hbm.at[pt_v.at[0]], sem3)
            cp1.wait()
            w1 = pltpu.async_copy(row2_v, psum_hbm.at[pt_v.at[1]], sem3)
            w0.wait()
            w1.wait()
            return u

        lax.fori_loop(0, nh, pair_b, 0)

        @pl.when(ng > 2 * nh)
        def _():
            unpack(2 * nh, 0, remap=True)
            pltpu.async_copy(sums_sh.at[bk_v.at[0]], row_v, sem).wait()
            pltpu.sync_copy(row_v, psum_hbm.at[pt_v.at[0]])

        plsc.subcore_barrier()
        return t

    lax.fori_loop(0, _PASSES, pass_body, 0)


def _segment_pool(featm, seg):
    mesh = plsc.VectorSubcoreMesh(core_axis_name="c", subcore_axis_name="s")
    fn = functools.partial(
        pl.kernel,
        mesh=mesh,
        compiler_params=pltpu.CompilerParams(needs_layout_passes=False),
        out_type=[
            jax.ShapeDtypeStruct((_NPAD, 128), _F32),
        ],
        scratch_types=[
            pltpu.VMEM((_CH,), _I32),
            pltpu.VMEM((_STCAP,), _I32),
            pltpu.VMEM((2, _G), _I32),
            pltpu.VMEM((2, _G), _I32),
            pltpu.VMEM((_G, 128), _F32),
            pltpu.VMEM((_G, 128), _F32),
            pltpu.VMEM((_ZB, 128), _F32),
            pltpu.VMEM((48,), _I32),
            pltpu.VMEM((48,), _I32),
            pltpu.VMEM((48,), _I32),
            pltpu.VMEM_SHARED((_SROWS, 128), _F32),
            pltpu.SemaphoreType.DMA,
            pltpu.SemaphoreType.DMA,
            pltpu.SemaphoreType.DMA,
        ],
    )(_sc_body)
    return fn(featm, seg)


def kernel(im, coord, noise, W_sparse, b_sparse, init_sky, W_out, b_out):
    B, F, C, H, W = im.shape
    n = F * H * W
    assert n == _N

    c0 = coord[0]
    im_ft = jnp.transpose(im[:, :, :3], (0, 1, 3, 4, 2)).reshape(n, 3)

    cx = c0[:, 0].reshape(n // 128, 128)
    cy = c0[:, 1].reshape(n // 128, 128)
    cz = c0[:, 2].reshape(n // 128, 128)
    blk = 256
    minp = pl.pallas_call(
        _min_kernel,
        grid=(n // 128 // blk,),
        in_specs=[pl.BlockSpec((blk, 128), lambda i: (i, 0))] * 3,
        out_specs=pl.BlockSpec((8, 128), lambda i: (0, 0)),
        out_shape=jax.ShapeDtypeStruct((8, 128), _F32),
    )(cx, cy, cz)

    seg2d = pl.pallas_call(
        _seg_kernel,
        grid=(n // 128 // blk,),
        in_specs=[
            pl.BlockSpec((8, 128), lambda i: (0, 0)),
            pl.BlockSpec((blk, 128), lambda i: (i, 0)),
            pl.BlockSpec((blk, 128), lambda i: (i, 0)),
            pl.BlockSpec((blk, 128), lambda i: (i, 0)),
        ],
        out_specs=pl.BlockSpec((blk, 128), lambda i: (i, 0)),
        out_shape=jax.ShapeDtypeStruct((n // 128, 128), _I32),
    )(minp, cx, cy, cz)
    seg = seg2d.reshape(n)

    imf_p = jnp.concatenate([im_ft, jnp.zeros((_P, 3), _F32)], axis=0)
    c0_p = jnp.concatenate([c0, jnp.zeros((_P, 3), _F32)], axis=0)
    nblk = _NPAD // _P
    featm = pl.pallas_call(
        _pre_kernel,
        grid=(nblk,),
        in_specs=[
            pl.BlockSpec((_P, 3), lambda i: (i, 0)),
            pl.BlockSpec((1, 16), lambda i: (0, 0)),
            pl.BlockSpec((19, 64), lambda i: (0, 0)),
            pl.BlockSpec((64,), lambda i: (0,)),
        ],
        out_specs=pl.BlockSpec((_P, 128), lambda i: (i, 0)),
        out_shape=jax.ShapeDtypeStruct((_NPAD, 128), _F32),
    )(imf_p, noise, W_sparse, b_sparse)

    (psum,) = _segment_pool(featm, seg)

    out = pl.pallas_call(
        _post_kernel,
        grid=(n // _P,),
        in_specs=[
            pl.BlockSpec((_P, 3), lambda i: (i, 0)),
            pl.BlockSpec((_P, 3), lambda i: (i, 0)),
            pl.BlockSpec((1, 16), lambda i: (0, 0)),
            pl.BlockSpec((19, 64), lambda i: (0, 0)),
            pl.BlockSpec((64,), lambda i: (0,)),
            pl.BlockSpec((_P, 128), lambda i: (i, 0)),
            pl.BlockSpec((86, 3), lambda i: (0, 0)),
            pl.BlockSpec((3,), lambda i: (0,)),
            pl.BlockSpec((64,), lambda i: (0,)),
        ],
        out_specs=pl.BlockSpec((_P, 3), lambda i: (i, 0)),
        out_shape=jax.ShapeDtypeStruct((n, 3), _F32),
    )(c0_p, imf_p, noise, W_sparse, b_sparse, psum, W_out, b_out, init_sky)
    return out
